# Initial kernel scaffold; baseline (speedup 1.0000x reference)
import jax
import jax.numpy as jnp
from jax.experimental import pallas as pl


def kernel(x, edge_index, edge_weight, Wz, bz, Wr, br, Wh, bh, Wa, ba, ctx, Wf, bf):
    raise NotImplementedError("write your pallas kernel here")


# speedup vs baseline: 10.6654x; 10.6654x over previous
import functools

import jax
import jax.numpy as jnp
from jax import lax
from jax.experimental import pallas as pl
from jax.experimental.pallas import tpu as pltpu
from jax.experimental.pallas import tpu_sc as plsc

N = 50000
E = 800000
F_IN = 4
HID = 32
T = 8

NC = 2
NS = 16
NW = NC * NS
K = 128
CH = 196
EPW = K * CH
EPAD = NW * EPW
NP = 50048
RPS = NP // NS

_MESH = plsc.VectorSubcoreMesh(core_axis_name="c", subcore_axis_name="s")
_SC_PARAMS = pltpu.CompilerParams(use_tc_tiling_on_sc=False)


def _deg_body(dst_hbm, ew_hbm, zero_hbm, out_hbm, dst_v, ew_v, msg_v, acc_sh):
    c = lax.axis_index("c")
    s = lax.axis_index("s")
    wid = c * NS + s
    pltpu.sync_copy(zero_hbm, acc_sh.at[pl.ds(s * RPS, RPS)])
    plsc.subcore_barrier()

    def chunk(g, carry):
        base = wid * EPW + g * K
        pltpu.sync_copy(dst_hbm.at[pl.ds(base, K)], dst_v)
        pltpu.sync_copy(ew_hbm.at[pl.ds(base, K)], ew_v)
        for j16 in range(K // 16):
            e16 = ew_v[pl.ds(j16 * 16, 16)]
            for j in range(16):
                msg_v[j16 * 16 + j, :] = jnp.broadcast_to(e16[j], (16,))
        pltpu.sync_copy(msg_v, acc_sh.at[dst_v], add=True)
        return carry

    lax.fori_loop(0, CH, chunk, 0)
    plsc.subcore_barrier()
    pltpu.sync_copy(acc_sh.at[pl.ds(s * RPS, RPS)],
                    out_hbm.at[pl.ds(c * NP + s * RPS, RPS)])


_deg_call = functools.partial(
    pl.kernel,
    out_type=jax.ShapeDtypeStruct((NC * NP, 16), jnp.float32),
    mesh=_MESH,
    scratch_types=[
        pltpu.VMEM((K,), jnp.int32),
        pltpu.VMEM((K,), jnp.float32),
        pltpu.VMEM((K, 16), jnp.float32),
        pltpu.VMEM_SHARED((NP, 16), jnp.float32),
    ],
    compiler_params=_SC_PARAMS,
)(_deg_body)


def _spmm_body(xs_hbm, src_hbm, dst_hbm, ew_hbm, zero_hbm, out_hbm,
               src_v, dst_v, ew_v, rows_v, acc_sh, sem):
    c = lax.axis_index("c")
    s = lax.axis_index("s")
    wid = c * NS + s
    pltpu.sync_copy(zero_hbm, acc_sh.at[pl.ds(s * RPS, RPS)])
    plsc.subcore_barrier()

    def chunk(g, carry):
        base = wid * EPW + g * K
        pltpu.sync_copy(src_hbm.at[pl.ds(base, K)], src_v)
        pltpu.sync_copy(dst_hbm.at[pl.ds(base, K)], dst_v)
        pltpu.sync_copy(ew_hbm.at[pl.ds(base, K)], ew_v)
        pltpu.async_copy(xs_hbm.at[src_v], rows_v, sem).wait()
        for j16 in range(K // 16):
            e16 = ew_v[pl.ds(j16 * 16, 16)]
            for j in range(16):
                e = e16[j]
                row = j16 * 16 + j
                rows_v[row, pl.ds(0, 16)] = rows_v[row, pl.ds(0, 16)] * e
                rows_v[row, pl.ds(16, 16)] = rows_v[row, pl.ds(16, 16)] * e
        pltpu.sync_copy(rows_v, acc_sh.at[dst_v], add=True)
        return carry

    lax.fori_loop(0, CH, chunk, 0)
    plsc.subcore_barrier()
    pltpu.sync_copy(acc_sh.at[pl.ds(s * RPS, RPS)],
                    out_hbm.at[pl.ds(c * NP + s * RPS, RPS)])


_spmm_call = functools.partial(
    pl.kernel,
    out_type=jax.ShapeDtypeStruct((NC * NP, HID), jnp.float32),
    mesh=_MESH,
    scratch_types=[
        pltpu.VMEM((K,), jnp.int32),
        pltpu.VMEM((K,), jnp.int32),
        pltpu.VMEM((K,), jnp.float32),
        pltpu.VMEM((K, HID), jnp.float32),
        pltpu.VMEM_SHARED((NP, HID), jnp.float32),
        pltpu.SemaphoreType.DMA,
    ],
    compiler_params=_SC_PARAMS,
)(_spmm_body)


_B = 2000
_G = N // _B

def _row_spec(w):
    return pl.BlockSpec((_B, w), lambda i: (i, 0))

def _full_spec(r, c):
    return pl.BlockSpec((r, c), lambda i: (0, 0))


def _prep_body(deg0, deg1, xflat, dis, dis2, xs):
    d = deg0[:, 0:1] + deg1[:, 0:1] + 1.0
    di = lax.rsqrt(d)
    dis[:, :] = di
    dis2[:, :] = 1.0 / d
    xs[:, :] = xflat[:, :] * di


def _prep(deg0, deg1, xflat):
    return pl.pallas_call(
        _prep_body,
        grid=(_G,),
        in_specs=[_row_spec(16), _row_spec(16), _row_spec(32)],
        out_specs=[_row_spec(1), _row_spec(1), _row_spec(32)],
        out_shape=[jax.ShapeDtypeStruct((N, 1), jnp.float32),
                   jax.ShapeDtypeStruct((N, 1), jnp.float32),
                   jax.ShapeDtypeStruct((N, HID), jnp.float32)],
    )(deg0, deg1, xflat)


def _axt(ax0, ax1, xcols, dis, dis2):
    return dis[:, :] * (ax0[:, :] + ax1[:, :]) + dis2[:, :] * xcols[:, :]


def _t0_body(ax0, ax1, xcols, dis, dis2, Wzx, bz, Whx, bh, h, hs):
    AX0 = _axt(ax0, ax1, xcols, dis, dis2)
    z = jax.nn.sigmoid(jnp.dot(AX0, Wzx[:, :], preferred_element_type=jnp.float32) + bz[:, :])
    hc = jnp.tanh(jnp.dot(AX0, Whx[:, :], preferred_element_type=jnp.float32) + bh[:, :])
    hn = (1.0 - z) * hc
    h[:, :] = hn
    hs[:, :] = hn * dis[:, :]


def _t0(ax0, ax1, xcols, dis, dis2, Wzx, bz, Whx, bh):
    return pl.pallas_call(
        _t0_body,
        grid=(_G,),
        in_specs=[_row_spec(4), _row_spec(4), _row_spec(4), _row_spec(1), _row_spec(1),
                  _full_spec(4, 32), _full_spec(1, 32), _full_spec(4, 32), _full_spec(1, 32)],
        out_specs=[_row_spec(32), _row_spec(32)],
        out_shape=[jax.ShapeDtypeStruct((N, HID), jnp.float32),
                   jax.ShapeDtypeStruct((N, HID), jnp.float32)],
    )(ax0, ax1, xcols, dis, dis2, Wzx, bz, Whx, bh)


def _gates_body(p0, p1, h, ax0, ax1, xcols, dis, dis2,
                Wzx, Wzh, bz, Wrx, Wrh, br, z_o, rh_o, rhs_o):
    di = dis[:, :]
    P = di * (p0[:, :] + p1[:, :]) + dis2[:, :] * h[:, :]
    AXt = _axt(ax0, ax1, xcols, dis, dis2)
    z = jax.nn.sigmoid(jnp.dot(AXt, Wzx[:, :], preferred_element_type=jnp.float32)
                       + jnp.dot(P, Wzh[:, :], preferred_element_type=jnp.float32) + bz[:, :])
    r = jax.nn.sigmoid(jnp.dot(AXt, Wrx[:, :], preferred_element_type=jnp.float32)
                       + jnp.dot(P, Wrh[:, :], preferred_element_type=jnp.float32) + br[:, :])
    rh = r * h[:, :]
    z_o[:, :] = z
    rh_o[:, :] = rh
    rhs_o[:, :] = rh * di


def _gates(p0, p1, h, ax0, ax1, xcols, dis, dis2, Wzx, Wzh, bz, Wrx, Wrh, br):
    return pl.pallas_call(
        _gates_body,
        grid=(_G,),
        in_specs=[_row_spec(32), _row_spec(32), _row_spec(32),
                  _row_spec(4), _row_spec(4), _row_spec(4), _row_spec(1), _row_spec(1),
                  _full_spec(4, 32), _full_spec(32, 32), _full_spec(1, 32),
                  _full_spec(4, 32), _full_spec(32, 32), _full_spec(1, 32)],
        out_specs=[_row_spec(32), _row_spec(32), _row_spec(32)],
        out_shape=[jax.ShapeDtypeStruct((N, HID), jnp.float32)] * 3,
    )(p0, p1, h, ax0, ax1, xcols, dis, dis2, Wzx, Wzh, bz, Wrx, Wrh, br)


def _update_body(q0, q1, rh, z, h, ax0, ax1, xcols, dis, dis2,
                 Whx, Whh, bh, h_o, hs_o):
    di = dis[:, :]
    Q = di * (q0[:, :] + q1[:, :]) + dis2[:, :] * rh[:, :]
    AXt = _axt(ax0, ax1, xcols, dis, dis2)
    hc = jnp.tanh(jnp.dot(AXt, Whx[:, :], preferred_element_type=jnp.float32)
                  + jnp.dot(Q, Whh[:, :], preferred_element_type=jnp.float32) + bh[:, :])
    zz = z[:, :]
    hn = zz * h[:, :] + (1.0 - zz) * hc
    h_o[:, :] = hn
    hs_o[:, :] = hn * di


def _update(q0, q1, rh, z, h, ax0, ax1, xcols, dis, dis2, Whx, Whh, bh):
    return pl.pallas_call(
        _update_body,
        grid=(_G,),
        in_specs=[_row_spec(32), _row_spec(32), _row_spec(32), _row_spec(32), _row_spec(32),
                  _row_spec(4), _row_spec(4), _row_spec(4), _row_spec(1), _row_spec(1),
                  _full_spec(4, 32), _full_spec(32, 32), _full_spec(1, 32)],
        out_specs=[_row_spec(32), _row_spec(32)],
        out_shape=[jax.ShapeDtypeStruct((N, HID), jnp.float32)] * 2,
    )(q0, q1, rh, z, h, ax0, ax1, xcols, dis, dis2, Whx, Whh, bh)


def _attn_body(*refs):
    hs = refs[:T]
    Wa, ba, ctxT, WfT, bf = refs[T:T + 5]
    out = refs[T + 5]
    als = []
    for t in range(T):
        S = jnp.tanh(jnp.dot(hs[t][:, :], Wa[:, :], preferred_element_type=jnp.float32)
                     + ba[:, :])
        als.append(jnp.sum(S * ctxT[:, :], axis=1, keepdims=True))
    al = jnp.concatenate(als, axis=1)
    m = jnp.max(al, axis=1, keepdims=True)
    ex = jnp.exp(al - m)
    ssum = jnp.sum(ex, axis=1, keepdims=True)
    ctxv = jnp.zeros_like(hs[0][:, :])
    for t in range(T):
        ctxv = ctxv + (ex[:, t:t + 1] / ssum) * hs[t][:, :]
    out[:, :] = jnp.sum(ctxv * WfT[:, :], axis=1, keepdims=True) + bf[:, :]


def _attn(hs, Wa, ba_row, ctxT, WfT, bf_row):
    return pl.pallas_call(
        _attn_body,
        grid=(_G,),
        in_specs=[_row_spec(32)] * T + [_full_spec(32, 32), _full_spec(1, 32),
                                        _full_spec(1, 32), _full_spec(1, 32),
                                        _full_spec(1, 1)],
        out_specs=[_row_spec(1)],
        out_shape=[jax.ShapeDtypeStruct((N, 1), jnp.float32)],
    )(*hs, Wa, ba_row, ctxT, WfT, bf_row)[0]


def kernel(x, edge_index, edge_weight, Wz, bz, Wr, br, Wh, bh, Wa, ba, ctx, Wf, bf):
    pad = EPAD - E
    src = jnp.concatenate([edge_index[0], jnp.zeros((pad,), jnp.int32)])
    dst = jnp.concatenate([edge_index[1], jnp.zeros((pad,), jnp.int32)])
    ew = jnp.concatenate([edge_weight, jnp.zeros((pad,), jnp.float32)])
    zeros16 = jnp.zeros((RPS, 16), jnp.float32)
    zeros32 = jnp.zeros((RPS, HID), jnp.float32)
    xflat = jnp.transpose(x, (0, 2, 1)).reshape(N, T * F_IN)

    Wzx, Wzh = Wz[:F_IN], Wz[F_IN:]
    Wrx, Wrh = Wr[:F_IN], Wr[F_IN:]
    Whx, Whh = Wh[:F_IN], Wh[F_IN:]
    bz_r = bz.reshape(1, HID)
    br_r = br.reshape(1, HID)
    bh_r = bh.reshape(1, HID)
    ba_r = ba.reshape(1, HID)
    ctxT = ctx.reshape(1, HID)
    WfT = Wf.reshape(1, HID)
    bf_r = bf.reshape(1, 1)

    degacc = _deg_call(dst, ew, zeros16)
    dis, dis2, xs = _prep(degacc[:N], degacc[NP:NP + N], xflat)

    accx = _spmm_call(xs, src, dst, ew, zeros32)
    ax0, ax1 = accx[:N], accx[NP:NP + N]

    def xcols(t):
        return xflat[:, t * F_IN:(t + 1) * F_IN]

    def axcols(t):
        return (ax0[:, t * F_IN:(t + 1) * F_IN], ax1[:, t * F_IN:(t + 1) * F_IN])

    a0, a1 = axcols(0)
    h, hs = _t0(a0, a1, xcols(0), dis, dis2, Wzx, bz_r, Whx, bh_r)
    hidden = [h]
    for t in range(1, T):
        a0, a1 = axcols(t)
        accp = _spmm_call(hs, src, dst, ew, zeros32)
        z, rh, rhs = _gates(accp[:N], accp[NP:NP + N], h, a0, a1, xcols(t), dis, dis2,
                            Wzx, Wzh, bz_r, Wrx, Wrh, br_r)
        accq = _spmm_call(rhs, src, dst, ew, zeros32)
        h, hs = _update(accq[:N], accq[NP:NP + N], rh, z, h, a0, a1, xcols(t), dis, dis2,
                        Whx, Whh, bh_r)
        hidden.append(h)

    return _attn(hidden, Wa, ba_r, ctxT, WfT, bf_r)


# speedup vs baseline: 18.2521x; 1.7113x over previous
import functools

import jax
import jax.numpy as jnp
from jax import lax
from jax.experimental import pallas as pl
from jax.experimental.pallas import tpu as pltpu
from jax.experimental.pallas import tpu_sc as plsc

N = 50000
E = 800000
F_IN = 4
HID = 32
T = 8

NC = 2
NS = 16
NW = NC * NS
K = 128
SK = 14
SCH = 14
CH = SK * SCH
EPW = K * CH
EPAD = NW * EPW
NP = 50048
RPS = NP // NS

_MESH = plsc.VectorSubcoreMesh(core_axis_name="c", subcore_axis_name="s")
_SC_PARAMS = pltpu.CompilerParams(use_tc_tiling_on_sc=False)


def _spmm_body(xs_hbm, srcr_hbm, dstr_hbm, ewr_hbm, zero_hbm, out_hbm,
               sb0, db0, wb0, sb1, db1, wb1, rows0, rows1, acc_sh, esem, gsem):
    c = lax.axis_index("c")
    s = lax.axis_index("s")
    wid = c * NS + s
    base_row = wid * SCH
    pltpu.sync_copy(zero_hbm, acc_sh.at[pl.ds(s * RPS, RPS)])
    pltpu.sync_copy(srcr_hbm.at[base_row], sb0)
    pltpu.sync_copy(dstr_hbm.at[base_row], db0)
    pltpu.sync_copy(ewr_hbm.at[base_row], wb0)
    plsc.subcore_barrier()

    def scale_scatter(rows, wb, db, k):
        for j16 in range(K // 16):
            e16 = wb[k, pl.ds(j16 * 16, 16)]
            for j in range(16):
                e = e16[j]
                row = j16 * 16 + j
                rows[row, pl.ds(0, 16)] = rows[row, pl.ds(0, 16)] * e
                rows[row, pl.ds(16, 16)] = rows[row, pl.ds(16, 16)] * e
        pltpu.sync_copy(rows, acc_sh.at[db.at[k]], add=True)

    def wait_rows(rows):
        pltpu.make_async_copy(xs_hbm.at[pl.ds(0, K)], rows, gsem).wait()

    def super_body(G, sb, db, wb, sbn, dbn, wbn):
        nxt = base_row + jnp.minimum(G + 1, SCH - 1)
        pltpu.async_copy(srcr_hbm.at[nxt], sbn, esem)
        pltpu.async_copy(dstr_hbm.at[nxt], dbn, esem)
        pltpu.async_copy(ewr_hbm.at[nxt], wbn, esem)
        pltpu.async_copy(xs_hbm.at[sb.at[0]], rows0, gsem)

        def body(k2, carry):
            k = 2 * k2
            wait_rows(rows0)
            pltpu.async_copy(xs_hbm.at[sb.at[k + 1]], rows1, gsem)
            scale_scatter(rows0, wb, db, k)
            wait_rows(rows1)

            @pl.when(k2 < SK // 2 - 1)
            def _():
                pltpu.async_copy(xs_hbm.at[sb.at[k + 2]], rows0, gsem)

            scale_scatter(rows1, wb, db, k + 1)
            return carry

        lax.fori_loop(0, SK // 2, body, 0)
        pltpu.make_async_copy(srcr_hbm.at[base_row], sbn, esem).wait()
        pltpu.make_async_copy(dstr_hbm.at[base_row], dbn, esem).wait()
        pltpu.make_async_copy(ewr_hbm.at[base_row], wbn, esem).wait()

    def pair(i, carry):
        super_body(2 * i, sb0, db0, wb0, sb1, db1, wb1)
        super_body(2 * i + 1, sb1, db1, wb1, sb0, db0, wb0)
        return carry

    lax.fori_loop(0, SCH // 2, pair, 0)
    plsc.subcore_barrier()
    pltpu.sync_copy(acc_sh.at[pl.ds(s * RPS, RPS)],
                    out_hbm.at[pl.ds(c * NP + s * RPS, RPS)])


_spmm_call = functools.partial(
    pl.kernel,
    out_type=jax.ShapeDtypeStruct((NC * NP, HID), jnp.float32),
    mesh=_MESH,
    scratch_types=[
        pltpu.VMEM((SK, K), jnp.int32),
        pltpu.VMEM((SK, K), jnp.int32),
        pltpu.VMEM((SK, K), jnp.float32),
        pltpu.VMEM((SK, K), jnp.int32),
        pltpu.VMEM((SK, K), jnp.int32),
        pltpu.VMEM((SK, K), jnp.float32),
        pltpu.VMEM((K, HID), jnp.float32),
        pltpu.VMEM((K, HID), jnp.float32),
        pltpu.VMEM_SHARED((NP, HID), jnp.float32),
        pltpu.SemaphoreType.DMA,
        pltpu.SemaphoreType.DMA,
    ],
    compiler_params=_SC_PARAMS,
)(_spmm_body)


_B = 2000
_G = N // _B

def _row_spec(w):
    return pl.BlockSpec((_B, w), lambda i: (i, 0))

def _full_spec(r, c):
    return pl.BlockSpec((r, c), lambda i: (0, 0))


def _prep_body(deg0, deg1, xflat, dis, dis2, xs):
    d = deg0[:, 0:1] + deg1[:, 0:1] + 1.0
    di = lax.rsqrt(d)
    dis[:, :] = di
    dis2[:, :] = 1.0 / d
    xs[:, :] = xflat[:, :] * di


def _prep(deg0, deg1, xflat):
    return pl.pallas_call(
        _prep_body,
        grid=(_G,),
        in_specs=[_row_spec(32), _row_spec(32), _row_spec(32)],
        out_specs=[_row_spec(1), _row_spec(1), _row_spec(32)],
        out_shape=[jax.ShapeDtypeStruct((N, 1), jnp.float32),
                   jax.ShapeDtypeStruct((N, 1), jnp.float32),
                   jax.ShapeDtypeStruct((N, HID), jnp.float32)],
    )(deg0, deg1, xflat)


def _axt(ax0, ax1, xcols, dis, dis2):
    return dis[:, :] * (ax0[:, :] + ax1[:, :]) + dis2[:, :] * xcols[:, :]


def _t0_body(ax0, ax1, xcols, dis, dis2, Wzx, bz, Whx, bh, h, hs):
    AX0 = _axt(ax0, ax1, xcols, dis, dis2)
    z = jax.nn.sigmoid(jnp.dot(AX0, Wzx[:, :], preferred_element_type=jnp.float32) + bz[:, :])
    hc = jnp.tanh(jnp.dot(AX0, Whx[:, :], preferred_element_type=jnp.float32) + bh[:, :])
    hn = (1.0 - z) * hc
    h[:, :] = hn
    hs[:, :] = hn * dis[:, :]


def _t0(ax0, ax1, xcols, dis, dis2, Wzx, bz, Whx, bh):
    return pl.pallas_call(
        _t0_body,
        grid=(_G,),
        in_specs=[_row_spec(4), _row_spec(4), _row_spec(4), _row_spec(1), _row_spec(1),
                  _full_spec(4, 32), _full_spec(1, 32), _full_spec(4, 32), _full_spec(1, 32)],
        out_specs=[_row_spec(32), _row_spec(32)],
        out_shape=[jax.ShapeDtypeStruct((N, HID), jnp.float32),
                   jax.ShapeDtypeStruct((N, HID), jnp.float32)],
    )(ax0, ax1, xcols, dis, dis2, Wzx, bz, Whx, bh)


def _gates_body(p0, p1, h, ax0, ax1, xcols, dis, dis2,
                Wzx, Wzh, bz, Wrx, Wrh, br, z_o, rh_o, rhs_o):
    di = dis[:, :]
    P = di * (p0[:, :] + p1[:, :]) + dis2[:, :] * h[:, :]
    AXt = _axt(ax0, ax1, xcols, dis, dis2)
    z = jax.nn.sigmoid(jnp.dot(AXt, Wzx[:, :], preferred_element_type=jnp.float32)
                       + jnp.dot(P, Wzh[:, :], preferred_element_type=jnp.float32) + bz[:, :])
    r = jax.nn.sigmoid(jnp.dot(AXt, Wrx[:, :], preferred_element_type=jnp.float32)
                       + jnp.dot(P, Wrh[:, :], preferred_element_type=jnp.float32) + br[:, :])
    rh = r * h[:, :]
    z_o[:, :] = z
    rh_o[:, :] = rh
    rhs_o[:, :] = rh * di


def _gates(p0, p1, h, ax0, ax1, xcols, dis, dis2, Wzx, Wzh, bz, Wrx, Wrh, br):
    return pl.pallas_call(
        _gates_body,
        grid=(_G,),
        in_specs=[_row_spec(32), _row_spec(32), _row_spec(32),
                  _row_spec(4), _row_spec(4), _row_spec(4), _row_spec(1), _row_spec(1),
                  _full_spec(4, 32), _full_spec(32, 32), _full_spec(1, 32),
                  _full_spec(4, 32), _full_spec(32, 32), _full_spec(1, 32)],
        out_specs=[_row_spec(32), _row_spec(32), _row_spec(32)],
        out_shape=[jax.ShapeDtypeStruct((N, HID), jnp.float32)] * 3,
    )(p0, p1, h, ax0, ax1, xcols, dis, dis2, Wzx, Wzh, bz, Wrx, Wrh, br)


def _update_body(q0, q1, rh, z, h, ax0, ax1, xcols, dis, dis2,
                 Whx, Whh, bh, h_o, hs_o):
    di = dis[:, :]
    Q = di * (q0[:, :] + q1[:, :]) + dis2[:, :] * rh[:, :]
    AXt = _axt(ax0, ax1, xcols, dis, dis2)
    hc = jnp.tanh(jnp.dot(AXt, Whx[:, :], preferred_element_type=jnp.float32)
                  + jnp.dot(Q, Whh[:, :], preferred_element_type=jnp.float32) + bh[:, :])
    zz = z[:, :]
    hn = zz * h[:, :] + (1.0 - zz) * hc
    h_o[:, :] = hn
    hs_o[:, :] = hn * di


def _update(q0, q1, rh, z, h, ax0, ax1, xcols, dis, dis2, Whx, Whh, bh):
    return pl.pallas_call(
        _update_body,
        grid=(_G,),
        in_specs=[_row_spec(32), _row_spec(32), _row_spec(32), _row_spec(32), _row_spec(32),
                  _row_spec(4), _row_spec(4), _row_spec(4), _row_spec(1), _row_spec(1),
                  _full_spec(4, 32), _full_spec(32, 32), _full_spec(1, 32)],
        out_specs=[_row_spec(32), _row_spec(32)],
        out_shape=[jax.ShapeDtypeStruct((N, HID), jnp.float32)] * 2,
    )(q0, q1, rh, z, h, ax0, ax1, xcols, dis, dis2, Whx, Whh, bh)


def _attn_body(*refs):
    hs = refs[:T]
    Wa, ba, ctxT, WfT, bf = refs[T:T + 5]
    out = refs[T + 5]
    als = []
    for t in range(T):
        S = jnp.tanh(jnp.dot(hs[t][:, :], Wa[:, :], preferred_element_type=jnp.float32)
                     + ba[:, :])
        als.append(jnp.sum(S * ctxT[:, :], axis=1, keepdims=True))
    al = jnp.concatenate(als, axis=1)
    m = jnp.max(al, axis=1, keepdims=True)
    ex = jnp.exp(al - m)
    ssum = jnp.sum(ex, axis=1, keepdims=True)
    ctxv = jnp.zeros_like(hs[0][:, :])
    for t in range(T):
        ctxv = ctxv + (ex[:, t:t + 1] / ssum) * hs[t][:, :]
    out[:, :] = jnp.sum(ctxv * WfT[:, :], axis=1, keepdims=True) + bf[:, :]


def _attn(hs, Wa, ba_row, ctxT, WfT, bf_row):
    return pl.pallas_call(
        _attn_body,
        grid=(_G,),
        in_specs=[_row_spec(32)] * T + [_full_spec(32, 32), _full_spec(1, 32),
                                        _full_spec(1, 32), _full_spec(1, 32),
                                        _full_spec(1, 1)],
        out_specs=[_row_spec(1)],
        out_shape=[jax.ShapeDtypeStruct((N, 1), jnp.float32)],
    )(*hs, Wa, ba_row, ctxT, WfT, bf_row)[0]


def kernel(x, edge_index, edge_weight, Wz, bz, Wr, br, Wh, bh, Wa, ba, ctx, Wf, bf):
    pad = EPAD - E
    srcr = jnp.concatenate([edge_index[0], jnp.zeros((pad,), jnp.int32)]).reshape(NW * SCH, SK, K)
    dstr = jnp.concatenate([edge_index[1], jnp.zeros((pad,), jnp.int32)]).reshape(NW * SCH, SK, K)
    ewr = jnp.concatenate([edge_weight, jnp.zeros((pad,), jnp.float32)]).reshape(NW * SCH, SK, K)
    zeros32 = jnp.zeros((RPS, HID), jnp.float32)
    ones32 = jnp.ones((N, HID), jnp.float32)
    xflat = jnp.transpose(x, (0, 2, 1)).reshape(N, T * F_IN)

    Wzx, Wzh = Wz[:F_IN], Wz[F_IN:]
    Wrx, Wrh = Wr[:F_IN], Wr[F_IN:]
    Whx, Whh = Wh[:F_IN], Wh[F_IN:]
    bz_r = bz.reshape(1, HID)
    br_r = br.reshape(1, HID)
    bh_r = bh.reshape(1, HID)
    ba_r = ba.reshape(1, HID)
    ctxT = ctx.reshape(1, HID)
    WfT = Wf.reshape(1, HID)
    bf_r = bf.reshape(1, 1)

    degacc = _spmm_call(ones32, srcr, dstr, ewr, zeros32)
    dis, dis2, xs = _prep(degacc[:N], degacc[NP:NP + N], xflat)

    accx = _spmm_call(xs, srcr, dstr, ewr, zeros32)
    ax0, ax1 = accx[:N], accx[NP:NP + N]

    def xcols(t):
        return xflat[:, t * F_IN:(t + 1) * F_IN]

    def axcols(t):
        return (ax0[:, t * F_IN:(t + 1) * F_IN], ax1[:, t * F_IN:(t + 1) * F_IN])

    a0, a1 = axcols(0)
    h, hs = _t0(a0, a1, xcols(0), dis, dis2, Wzx, bz_r, Whx, bh_r)
    hidden = [h]
    for t in range(1, T):
        a0, a1 = axcols(t)
        accp = _spmm_call(hs, srcr, dstr, ewr, zeros32)
        z, rh, rhs = _gates(accp[:N], accp[NP:NP + N], h, a0, a1, xcols(t), dis, dis2,
                            Wzx, Wzh, bz_r, Wrx, Wrh, br_r)
        accq = _spmm_call(rhs, srcr, dstr, ewr, zeros32)
        h, hs = _update(accq[:N], accq[NP:NP + N], rh, z, h, a0, a1, xcols(t), dis, dis2,
                        Whx, Whh, bh_r)
        hidden.append(h)

    return _attn(hidden, Wa, ba_r, ctxT, WfT, bf_r)
